# baseline (speedup 1.0000x reference)
import functools

import numpy as np
import jax
import jax.numpy as jnp
from jax import lax
from jax.experimental import pallas as pl
from jax.experimental.pallas import tpu as pltpu
from jax.experimental.pallas import tpu_sc as plsc

H = 224
W = 224
N = H * W
NCHUNK = 4
Q = N // NCHUNK
GRP = 128
LANES = 16
BLKW = 32


def _zz_perm(h, w):
    idx = []
    for s in range(h + w - 1):
        if s % 2 == 0:
            for i in range(min(s, h - 1), max(0, s - w + 1) - 1, -1):
                j = s - i
                if j < w:
                    idx.append(i * w + j)
        else:
            for i in range(max(0, s - w + 1), min(s, h - 1) + 1):
                j = s - i
                if j < w:
                    idx.append(i * w + j)
    return np.array(idx, dtype=np.int32)


@functools.cache
def _tables():
    perm = _zz_perm(H, W)
    blk_lists, src_lists = [], []
    for q in range(NCHUNK):
        src = perm[q * Q:(q + 1) * Q]
        blk_lists.append(np.unique(src // BLKW))
        src_lists.append(src)
    nblk = max(len(b) for b in blk_lists)
    ngrp = -(-nblk // GRP)
    nslot = ngrp * GRP * BLKW
    blk = np.zeros((NCHUNK, ngrp, GRP), dtype=np.int32)
    stbl = np.full((NCHUNK, nslot), Q, dtype=np.int32)
    for q in range(NCHUNK):
        blocks, src = blk_lists[q], src_lists[q]
        pad = np.full(ngrp * GRP - len(blocks), blocks[-1], dtype=np.int32)
        blk[q] = np.concatenate([blocks, pad]).reshape(ngrp, GRP)
        rank = np.zeros(N // BLKW, dtype=np.int32)
        rank[blocks] = np.arange(len(blocks), dtype=np.int32)
        slot = rank[src // BLKW] * BLKW + (src % BLKW)
        stbl[q, slot] = np.arange(Q, dtype=np.int32)
    return blk, stbl, ngrp


def _sc_reorder(x2, blk, stbl, rows, ngrp):
    info = plsc.get_sparse_core_info()
    nw = info.num_cores * info.num_subcores
    wpc = nw // NCHUNK
    ipw = rows // wpc
    blocks_per_img = N // BLKW
    nslot = ngrp * GRP * BLKW

    mesh = plsc.VectorSubcoreMesh(core_axis_name="c", subcore_axis_name="s")

    @functools.partial(
        pl.kernel,
        out_type=jax.ShapeDtypeStruct((rows, N), jnp.float32),
        mesh=mesh,
        compiler_params=pltpu.CompilerParams(
            needs_layout_passes=False, use_tc_tiling_on_sc=False),
        scratch_types=[
            pltpu.VMEM((ngrp, GRP), jnp.int32),
            pltpu.VMEM((ngrp, GRP), jnp.int32),
            pltpu.VMEM((ngrp, GRP), jnp.int32),
            pltpu.VMEM((28672,), jnp.float32),
            pltpu.VMEM((28672,), jnp.float32),
            pltpu.VMEM((nslot,), jnp.int32),
            pltpu.VMEM((Q + LANES,), jnp.float32),
            pltpu.VMEM((Q + LANES,), jnp.float32),
            pltpu.SemaphoreType.DMA,
            pltpu.SemaphoreType.DMA,
            pltpu.SemaphoreType.DMA,
        ],
    )
    def zz(x_hbm, blk_hbm, stbl_hbm, out_hbm, blk_v, blkadj0, blkadj1,
           staged0, staged1, stbl_v, out0, out1, gsem0, gsem1, osem):
        cid = lax.axis_index("c")
        sid = lax.axis_index("s")
        wid = sid * info.num_cores + cid
        chunk = wid % NCHUNK
        img0 = (wid // NCHUNK) * ipw
        pltpu.sync_copy(blk_hbm.at[chunk], blk_v)
        pltpu.sync_copy(stbl_hbm.at[chunk], stbl_v)
        qoff = chunk * Q

        def fire(img, adj, stg, sem):
            imgc = jnp.minimum(img, rows - 1)
            pltpu.async_copy(x_hbm.at[imgc, pl.ds(0, 28672)], stg, sem)

        def drain_gather(stg, sem):
            pltpu.make_async_copy(
                x_hbm.at[0, pl.ds(0, 28672)], stg, sem).wait()

        def shuffle(stg, out_v):
            @plsc.parallel_loop(0, ngrp * GRP, 1, unroll=8)
            def _(r):
                for h in range(BLKW // LANES):
                    vals = stg[r, pl.ds(h * LANES, LANES)]
                    sidx = stbl_v[pl.ds(r * BLKW + h * LANES, LANES)]
                    plsc.store_scatter(out_v, [sidx], vals)

        def put(out_v, img):
            pltpu.async_copy(out_v.at[pl.ds(0, Q)],
                             out_hbm.at[img, pl.ds(qoff, Q)], osem)

        def drain_put():
            pltpu.make_async_copy(out_hbm.at[0, pl.ds(qoff, Q)],
                                  out0.at[pl.ds(0, Q)], osem).wait()

        fire(img0, blkadj0, staged0, gsem0)

        def pair(u, carry):
            img = img0 + 2 * u
            fire(img + 1, blkadj1, staged1, gsem1)
            drain_gather(staged0, gsem0)
            fire(img + 2, blkadj0, staged0, gsem0)
            drain_gather(staged1, gsem1)
            return carry

        lax.fori_loop(0, ipw // 2, pair, 0)
        drain_gather(staged0, gsem0)

    return zz(x2, blk, stbl)


def kernel(x):
    B, C, h, w = x.shape
    rows = B * C
    blk_np, stbl_np, ngrp = _tables()
    x2 = x.reshape(rows, N)
    out = _sc_reorder(x2, jnp.asarray(blk_np), jnp.asarray(stbl_np), rows, ngrp)
    return out.reshape(B, C, h, w)


# speedup vs baseline: 1.0123x; 1.0123x over previous
import functools

import numpy as np
import jax
import jax.numpy as jnp
from jax import lax
from jax.experimental import pallas as pl
from jax.experimental.pallas import tpu as pltpu
from jax.experimental.pallas import tpu_sc as plsc

H = 224
W = 224
N = H * W
NPHASE = 8
Q = N // NPHASE
NBAND = 4
BROWS = H // NBAND
LANES = 16


def _zz_perm(h, w):
    idx = []
    for s in range(h + w - 1):
        if s % 2 == 0:
            for i in range(min(s, h - 1), max(0, s - w + 1) - 1, -1):
                j = s - i
                if j < w:
                    idx.append(i * w + j)
        else:
            for i in range(max(0, s - w + 1), min(s, h - 1) + 1):
                j = s - i
                if j < w:
                    idx.append(i * w + j)
    return np.array(idx, dtype=np.int32)


@functools.cache
def _tables():
    perm = _zz_perm(H, W)
    bands_needed = []
    min_row = []
    for q in range(NPHASE):
        src = perm[q * Q:(q + 1) * Q]
        rows = src // W
        bands_needed.append(int(rows.max()) // BROWS + 1)
        min_row.append(int(rows.min()))
    free_after = []
    for q in range(NPHASE):
        later_min = min(min_row[q + 1:], default=H)
        free_after.append(later_min >= BROWS)
    return perm, bands_needed, free_after


def _sc_reorder(x2, perm, rows):
    info = plsc.get_sparse_core_info()
    nw = info.num_cores * info.num_subcores
    ipw = rows // nw

    mesh = plsc.VectorSubcoreMesh(core_axis_name="c", subcore_axis_name="s")
    _, bands_needed, free_after = _tables()
    pre0 = next(q for q in range(NPHASE) if free_after[q])

    @functools.partial(
        pl.kernel,
        out_type=jax.ShapeDtypeStruct((rows, N), jnp.float32),
        mesh=mesh,
        compiler_params=pltpu.CompilerParams(
            needs_layout_passes=False, use_tc_tiling_on_sc=False),
        scratch_types=[
            pltpu.VMEM((N,), jnp.float32),
            pltpu.VMEM((N,), jnp.int32),
            pltpu.VMEM((Q,), jnp.float32),
            pltpu.VMEM((Q,), jnp.float32),
            pltpu.SemaphoreType.DMA,
            pltpu.SemaphoreType.DMA,
        ],
    )
    def zz(x_hbm, perm_hbm, out_hbm, staged, tbl_v, outa, outb, gsem, osem):
        cid = lax.axis_index("c")
        sid = lax.axis_index("s")
        wid = sid * info.num_cores + cid
        img0 = wid * ipw
        pltpu.sync_copy(perm_hbm, tbl_v)
        outs = [outa, outb]

        def fire_band(img, b):
            sl = pl.ds(b * BROWS * W, BROWS * W)
            pltpu.async_copy(x_hbm.at[img, sl], staged.at[sl], gsem)

        def wait_band(b):
            sl = pl.ds(b * BROWS * W, BROWS * W)
            pltpu.make_async_copy(
                x_hbm.at[0, sl], staged.at[sl], gsem).wait()

        def drain_put(q):
            sl = pl.ds(q * Q, Q)
            pltpu.make_async_copy(
                out_hbm.at[0, sl], outs[q % 2], osem).wait()

        def per_image(t, carry):
            img = img0 + t
            nxt = jnp.minimum(img + 1, rows - 1)
            for b in range(1, NBAND):
                fire_band(img, b)
            waited = 0
            for q in range(NPHASE):
                for b in range(waited, bands_needed[q]):
                    wait_band(b)
                waited = max(waited, bands_needed[q])
                if q >= 2:
                    drain_put(q - 2)
                out_v = outs[q % 2]

                @plsc.parallel_loop(0, Q // LANES, 1, unroll=8)
                def _(k):
                    gidx = tbl_v[pl.ds(q * Q + k * LANES, LANES)]
                    out_v[pl.ds(k * LANES, LANES)] = plsc.load_gather(
                        staged, [gidx])

                pltpu.async_copy(out_v, out_hbm.at[img, pl.ds(q * Q, Q)],
                                 osem)
                if q == pre0:
                    fire_band(nxt, 0)
            drain_put(NPHASE - 2)
            drain_put(NPHASE - 1)
            return carry

        fire_band(img0, 0)
        lax.fori_loop(0, ipw, per_image, 0)
        wait_band(0)

    return zz(x2, perm)


def kernel(x):
    B, C, h, w = x.shape
    rows = B * C
    perm_np, _, _ = _tables()
    x2 = x.reshape(rows, N)
    out = _sc_reorder(x2, jnp.asarray(perm_np), rows)
    return out.reshape(B, C, h, w)


# speedup vs baseline: 1.0224x; 1.0099x over previous
import functools

import numpy as np
import jax
import jax.numpy as jnp
from jax import lax
from jax.experimental import pallas as pl
from jax.experimental.pallas import tpu as pltpu
from jax.experimental.pallas import tpu_sc as plsc

H = 224
W = 224
N = H * W
NPHASE = 8
Q = N // NPHASE
NBAND = 4
BROWS = H // NBAND
LANES = 16


def _zz_perm(h, w):
    idx = []
    for s in range(h + w - 1):
        if s % 2 == 0:
            for i in range(min(s, h - 1), max(0, s - w + 1) - 1, -1):
                j = s - i
                if j < w:
                    idx.append(i * w + j)
        else:
            for i in range(max(0, s - w + 1), min(s, h - 1) + 1):
                j = s - i
                if j < w:
                    idx.append(i * w + j)
    return np.array(idx, dtype=np.int32)


@functools.cache
def _tables():
    perm = _zz_perm(H, W)
    min_row, max_row = [], []
    for q in range(NPHASE):
        rows = perm[q * Q:(q + 1) * Q] // W
        min_row.append(int(rows.min()))
        max_row.append(int(rows.max()))
    suffix_min = [min(min_row[q + 1:], default=H) for q in range(NPHASE)]
    b1 = suffix_min[NPHASE - 3]
    b2 = suffix_min[NPHASE - 2]
    bb = [0, b1, b2, (b2 + H) // 2, H]
    assert all(bb[i] < bb[i + 1] for i in range(NBAND))
    bands_needed = [next(i for i in range(NBAND) if max_row[q] < bb[i + 1]) + 1
                    for q in range(NPHASE)]
    prefetch_at = []
    for b in range(NBAND):
        ok = [q for q in range(NPHASE) if suffix_min[q] >= bb[b + 1]]
        prefetch_at.append(ok[0] if ok else None)
    return perm, bb, bands_needed, prefetch_at


def _sc_reorder(x2, perm, rows):
    info = plsc.get_sparse_core_info()
    nw = info.num_cores * info.num_subcores
    ipw = rows // nw

    mesh = plsc.VectorSubcoreMesh(core_axis_name="c", subcore_axis_name="s")
    _, bb, bands_needed, prefetch_at = _tables()
    late_bands = [b for b in range(NBAND) if prefetch_at[b] is None]
    early_bands = [b for b in range(NBAND) if prefetch_at[b] is not None]

    @functools.partial(
        pl.kernel,
        out_type=jax.ShapeDtypeStruct((rows, N), jnp.float32),
        mesh=mesh,
        compiler_params=pltpu.CompilerParams(
            needs_layout_passes=False, use_tc_tiling_on_sc=False),
        scratch_types=[
            pltpu.VMEM((N,), jnp.float32),
            pltpu.VMEM((N,), jnp.int32),
            pltpu.VMEM((Q,), jnp.float32),
            pltpu.VMEM((Q,), jnp.float32),
            pltpu.SemaphoreType.DMA,
            pltpu.SemaphoreType.DMA,
        ],
    )
    def zz(x_hbm, perm_hbm, out_hbm, staged, tbl_v, outa, outb, gsem, osem):
        cid = lax.axis_index("c")
        sid = lax.axis_index("s")
        wid = sid * info.num_cores + cid
        img0 = wid * ipw
        pltpu.sync_copy(perm_hbm, tbl_v)
        outs = [outa, outb]

        def fire_band(img, b):
            sl = pl.ds(bb[b] * W, (bb[b + 1] - bb[b]) * W)
            pltpu.async_copy(x_hbm.at[img, sl], staged.at[sl], gsem)

        def wait_band(b):
            sl = pl.ds(bb[b] * W, (bb[b + 1] - bb[b]) * W)
            pltpu.make_async_copy(
                x_hbm.at[0, sl], staged.at[sl], gsem).wait()

        def drain_put(q):
            sl = pl.ds(q * Q, Q)
            pltpu.make_async_copy(
                out_hbm.at[0, sl], outs[q % 2], osem).wait()

        def per_image(t, carry):
            img = img0 + t
            nxt = jnp.minimum(img + 1, rows - 1)
            for b in late_bands:
                fire_band(img, b)
            waited = 0
            for q in range(NPHASE):
                for b in range(waited, bands_needed[q]):
                    wait_band(b)
                waited = max(waited, bands_needed[q])
                if q >= 2:
                    drain_put(q - 2)
                out_v = outs[q % 2]

                @plsc.parallel_loop(0, Q // LANES, 1, unroll=8)
                def _(k):
                    gidx = tbl_v[pl.ds(q * Q + k * LANES, LANES)]
                    out_v[pl.ds(k * LANES, LANES)] = plsc.load_gather(
                        staged, [gidx])

                pltpu.async_copy(out_v, out_hbm.at[img, pl.ds(q * Q, Q)],
                                 osem)
                for b in early_bands:
                    if prefetch_at[b] == q:
                        fire_band(nxt, b)
            drain_put(NPHASE - 2)
            drain_put(NPHASE - 1)
            return carry

        for b in early_bands:
            fire_band(img0, b)
        lax.fori_loop(0, ipw, per_image, 0)
        for b in early_bands:
            wait_band(b)

    return zz(x2, perm)


def kernel(x):
    B, C, h, w = x.shape
    rows = B * C
    perm_np = _tables()[0]
    x2 = x.reshape(rows, N)
    out = _sc_reorder(x2, jnp.asarray(perm_np), rows)
    return out.reshape(B, C, h, w)


# speedup vs baseline: 1.0249x; 1.0024x over previous
import functools

import numpy as np
import jax
import jax.numpy as jnp
from jax import lax
from jax.experimental import pallas as pl
from jax.experimental.pallas import tpu as pltpu
from jax.experimental.pallas import tpu_sc as plsc

H = 224
W = 224
N = H * W
NPHASE = 8
Q = N // NPHASE
NBAND = 4
LANES = 16


def _zz_perm(h, w):
    idx = []
    for s in range(h + w - 1):
        if s % 2 == 0:
            for i in range(min(s, h - 1), max(0, s - w + 1) - 1, -1):
                j = s - i
                if j < w:
                    idx.append(i * w + j)
        else:
            for i in range(max(0, s - w + 1), min(s, h - 1) + 1):
                j = s - i
                if j < w:
                    idx.append(i * w + j)
    return np.array(idx, dtype=np.int32)


@functools.cache
def _tables():
    perm = _zz_perm(H, W)
    min_row, max_row = [], []
    for q in range(NPHASE):
        rows = perm[q * Q:(q + 1) * Q] // W
        min_row.append(int(rows.min()))
        max_row.append(int(rows.max()))
    suffix_min = [min(min_row[q + 1:], default=H) for q in range(NPHASE)]
    b1 = suffix_min[NPHASE - 3]
    b2 = suffix_min[NPHASE - 2]
    bb = [0, b1, b2, (b2 + H) // 2, H]
    assert all(bb[i] < bb[i + 1] for i in range(NBAND))
    bands_needed = [next(i for i in range(NBAND) if max_row[q] < bb[i + 1]) + 1
                    for q in range(NPHASE)]
    prefetch_at = []
    for b in range(NBAND):
        ok = [q for q in range(NPHASE) if suffix_min[q] >= bb[b + 1]]
        prefetch_at.append(ok[0] if ok else None)
    return perm, bb, bands_needed, prefetch_at


def _sc_reorder(x2, perm, rows):
    info = plsc.get_sparse_core_info()
    nw = info.num_cores * info.num_subcores
    ipw = rows // nw

    mesh = plsc.VectorSubcoreMesh(core_axis_name="c", subcore_axis_name="s")
    _, bb, bands_needed, prefetch_at = _tables()
    late_bands = [b for b in range(NBAND) if prefetch_at[b] is None]
    early_bands = [b for b in range(NBAND) if prefetch_at[b] is not None]

    @functools.partial(
        pl.kernel,
        out_type=jax.ShapeDtypeStruct((rows, N), jnp.float32),
        mesh=mesh,
        compiler_params=pltpu.CompilerParams(
            needs_layout_passes=False, use_tc_tiling_on_sc=False),
        scratch_types=[
            pltpu.VMEM((N,), jnp.float32),
            pltpu.VMEM((N,), jnp.int32),
            pltpu.VMEM((Q,), jnp.float32),
            pltpu.VMEM((Q,), jnp.float32),
            pltpu.SemaphoreType.DMA,
            pltpu.SemaphoreType.DMA,
        ],
    )
    def zz(x_hbm, perm_hbm, out_hbm, staged, tbl_v, outa, outb, gsem, osem):
        cid = lax.axis_index("c")
        sid = lax.axis_index("s")
        wid = sid * info.num_cores + cid
        img0 = wid * ipw
        pltpu.sync_copy(perm_hbm, tbl_v)
        outs = [outa, outb]

        def fire_band(img, b):
            sl = pl.ds(bb[b] * W, (bb[b + 1] - bb[b]) * W)
            pltpu.async_copy(x_hbm.at[img, sl], staged.at[sl], gsem)

        def wait_band(b):
            sl = pl.ds(bb[b] * W, (bb[b + 1] - bb[b]) * W)
            pltpu.make_async_copy(
                x_hbm.at[0, sl], staged.at[sl], gsem).wait()

        def drain_put(q):
            sl = pl.ds(q * Q, Q)
            pltpu.make_async_copy(
                out_hbm.at[0, sl], outs[q % 2], osem).wait()

        def per_image(t, carry):
            img = img0 + t
            nxt = jnp.minimum(img + 1, rows - 1)
            for b in late_bands:
                fire_band(img, b)
            waited = 0
            for q in range(NPHASE):
                for b in range(waited, bands_needed[q]):
                    wait_band(b)
                waited = max(waited, bands_needed[q])
                if q >= 2:
                    drain_put(q - 2)
                out_v = outs[q % 2]

                @plsc.parallel_loop(0, Q // LANES, 1, unroll=8)
                def _(k):
                    gidx = tbl_v[pl.ds(q * Q + k * LANES, LANES)]
                    out_v[pl.ds(k * LANES, LANES)] = plsc.load_gather(
                        staged, [gidx])

                pltpu.async_copy(out_v, out_hbm.at[img, pl.ds(q * Q, Q)],
                                 osem)
                for b in early_bands:
                    if prefetch_at[b] == q:
                        fire_band(nxt, b)
            drain_put(NPHASE - 2)
            drain_put(NPHASE - 1)
            return carry

        for b in early_bands:
            fire_band(img0, b)
        lax.fori_loop(0, ipw, per_image, 0)
        for b in early_bands:
            wait_band(b)

    return zz(x2, perm)


def kernel(x):
    B, C, h, w = x.shape
    rows = B * C
    perm_np = _tables()[0]
    x2 = x.reshape(rows, N)
    out = _sc_reorder(x2, jnp.asarray(perm_np), rows)
    return out.reshape(B, C, h, w)
